# Initial kernel scaffold; baseline (speedup 1.0000x reference)
import jax
import jax.numpy as jnp
from jax.experimental import pallas as pl


def kernel(orders, before_loss, after_loss, test_sample_ids, emb, W1, b1, W2, b2, W3, b3):
    raise NotImplementedError("write your pallas kernel here")


# speedup vs baseline: 7.7411x; 7.7411x over previous
import functools

import jax
import jax.numpy as jnp
from jax import lax
from jax.experimental import pallas as pl
from jax.experimental.pallas import tpu as pltpu
from jax.experimental.pallas import tpu_sc as plsc

_TRAIN_N = 10000
_BS = 4096
_FB = 50
_EMB = 16
_HYPER = 0.01


def _sc_gather_l2(table, idx_g):
    info = plsc.get_sparse_core_info()
    nw = info.num_cores * info.num_subcores
    b = idx_g.shape[0]
    b_per_w = b // nw
    rpw = b_per_w // 8
    v = table.shape[0]
    rows_per_w = v // nw
    n_chunks = 25
    l2_chunk = rows_per_w // n_chunks
    unroll = 10
    mesh = plsc.VectorSubcoreMesh(core_axis_name="c", subcore_axis_name="s")

    @functools.partial(
        pl.kernel,
        mesh=mesh,
        out_type=(
            jax.ShapeDtypeStruct((b // 8, 128), jnp.float32),
            jax.ShapeDtypeStruct((nw, _EMB), jnp.float32),
        ),
        scratch_types=[
            pltpu.VMEM((b_per_w,), jnp.int32),
            pltpu.VMEM((b_per_w, _EMB), jnp.float32),
            pltpu.VMEM((l2_chunk, _EMB), jnp.float32),
            pltpu.VMEM((_EMB,), jnp.float32),
            pltpu.SemaphoreType.DMA,
            pltpu.SemaphoreType.DMA,
        ],
        compiler_params=pltpu.CompilerParams(use_tc_tiling_on_sc=False),
    )
    def sc_kernel(idx_hbm, table_hbm, out_hbm, sq_hbm,
                  idx_v, rows_v, chunk_v, acc_v, sem, sem2):
        wid = lax.axis_index("s") * info.num_cores + lax.axis_index("c")
        base = wid * b_per_w
        pltpu.sync_copy(idx_hbm.at[pl.ds(base, b_per_w)], idx_v)
        pltpu.async_copy(table_hbm.at[idx_v], rows_v, sem).wait()
        wdescs = [
            pltpu.async_copy(
                rows_v.at[pl.ds(rpw * m, rpw)],
                out_hbm.at[pl.ds(wid * rpw, rpw), pl.ds(_EMB * m, _EMB)],
                sem2,
            )
            for m in range(8)
        ]
        for d in wdescs:
            d.wait()

        ebase = wid * rows_per_w

        def chunk_body(c, acc):
            pltpu.sync_copy(
                table_hbm.at[pl.ds(ebase + c * l2_chunk, l2_chunk)], chunk_v
            )

            def row_body(i, a):
                for k in range(unroll):
                    r = chunk_v[i * unroll + k]
                    a = a + r * r
                return a

            return lax.fori_loop(0, l2_chunk // unroll, row_body, acc)

        acc = lax.fori_loop(0, n_chunks, chunk_body, jnp.zeros((_EMB,), jnp.float32))
        acc_v[...] = acc
        pltpu.sync_copy(acc_v, sq_hbm.at[wid])

    return sc_kernel(idx_g, table)


def _tc_kernel(gath_ref, w1k_ref, b1r_ref, p2_ref, p3_ref, bsc_ref,
               before_ref, after_ref, sq_ref,
               pred_ref, mse_ref, l2_ref, tot_ref,
               acca_ref, accb_ref):
    j = pl.program_id(0)

    @pl.when(j == 0)
    def _init():
        acca_ref[...] = jnp.zeros_like(acca_ref)
        accb_ref[...] = jnp.zeros_like(accb_ref)

    x = gath_ref[...]
    h = jnp.dot(x, w1k_ref[...], preferred_element_type=jnp.float32) + b1r_ref[...]
    h = jnp.maximum(h, 0.0)
    acca_ref[...] += jnp.dot(h, p2_ref[...], preferred_element_type=jnp.float32)
    accb_ref[...] += jnp.dot(h, p3_ref[...], preferred_element_type=jnp.float32)

    @pl.when(j == pl.num_programs(0) - 1)
    def _finish():
        pa = acca_ref[...] + _FB * bsc_ref[0]
        pb = accb_ref[...] + _FB * bsc_ref[1]
        pred = pa * before_ref[...] + pb
        pred_ref[...] = pred
        diff = after_ref[...] - pred
        mse = jnp.sum(diff * diff) / _BS
        sq = sq_ref[...]
        s0 = jnp.sum(sq[:, 0:1])
        s1 = jnp.sum(sq[:, 1:2])
        l2 = _HYPER * (jnp.sqrt(s0) + jnp.sqrt(s1))
        mse_ref[0, 0] = mse
        l2_ref[0, 0] = l2
        tot_ref[0, 0] = mse + l2


def kernel(orders, before_loss, after_loss, test_sample_ids, emb, W1, b1, W2, b2, W3, b3):
    idx_flat = (_TRAIN_N * test_sample_ids[None, :] + orders.T).reshape(-1)
    idx_flat = idx_flat.astype(jnp.int32)
    idx_g = idx_flat.reshape(32, 800, 8).transpose(0, 2, 1).reshape(-1)

    gath128, sumsq = _sc_gather_l2(emb, idx_g)

    eye8 = jnp.eye(8, dtype=jnp.float32)
    w1k = jnp.kron(eye8, W1)
    b1r = jnp.tile(b1, 8).reshape(1, 800)
    p2 = jnp.kron(eye8, W2)
    p3 = jnp.kron(eye8, W3)
    bsc = jnp.concatenate([b2, b3]).astype(jnp.float32)

    n_steps = _FB
    rows_per_step = gath128.shape[0] // n_steps

    out = pl.pallas_call(
        _tc_kernel,
        grid=(n_steps,),
        in_specs=[
            pl.BlockSpec((rows_per_step, 128), lambda j: (j, 0)),
            pl.BlockSpec((128, 800), lambda j: (0, 0)),
            pl.BlockSpec((1, 800), lambda j: (0, 0)),
            pl.BlockSpec((800, 8), lambda j: (0, 0)),
            pl.BlockSpec((800, 8), lambda j: (0, 0)),
            pl.BlockSpec(memory_space=pltpu.SMEM),
            pl.BlockSpec((rows_per_step, 8), lambda j: (0, 0)),
            pl.BlockSpec((rows_per_step, 8), lambda j: (0, 0)),
            pl.BlockSpec((32, _EMB), lambda j: (0, 0)),
        ],
        out_specs=[
            pl.BlockSpec((rows_per_step, 8), lambda j: (0, 0)),
            pl.BlockSpec(memory_space=pltpu.SMEM),
            pl.BlockSpec(memory_space=pltpu.SMEM),
            pl.BlockSpec(memory_space=pltpu.SMEM),
        ],
        out_shape=[
            jax.ShapeDtypeStruct((rows_per_step, 8), jnp.float32),
            jax.ShapeDtypeStruct((1, 1), jnp.float32),
            jax.ShapeDtypeStruct((1, 1), jnp.float32),
            jax.ShapeDtypeStruct((1, 1), jnp.float32),
        ],
        scratch_shapes=[
            pltpu.VMEM((rows_per_step, 8), jnp.float32),
            pltpu.VMEM((rows_per_step, 8), jnp.float32),
        ],
    )(
        gath128, w1k, b1r, p2, p3, bsc,
        before_loss.reshape(rows_per_step, 8), after_loss.reshape(rows_per_step, 8),
        sumsq,
    )
    pred2, mse2, l22, tot2 = out
    return (mse2.reshape(()), l22.reshape(()), pred2.reshape(_BS), tot2.reshape(()))
